# Initial kernel scaffold; baseline (speedup 1.0000x reference)
import jax
import jax.numpy as jnp
from jax.experimental import pallas as pl


def kernel(field, grid_points, sample_positions):
    raise NotImplementedError("write your pallas kernel here")


# speedup vs baseline: 7000.5121x; 7000.5121x over previous
import functools

import jax
import jax.numpy as jnp
from jax import lax
from jax.experimental import pallas as pl
from jax.experimental.pallas import tpu as pltpu
from jax.experimental.pallas import tpu_sc as plsc

B, G, D, N = 16, 4096, 128, 16384
NC, NS, L = 2, 16, 16
NW = NC * NS
TOTAL = B * N
PER_W = TOTAL // NW
CHUNK = 128
NCHUNK = PER_W // CHUNK


def _sc_body(field_hbm, pos_hbm, out_hbm, pos_v, w_v, idx0_v, idx1_v,
             f0_v, f1_v, o_v, sem0, sem1):
    wid = lax.axis_index("s") * NC + lax.axis_index("c")
    wbase = wid * PER_W
    b_off = (wbase // N) * G

    def chunk_body(c, _):
        base = wbase + c * CHUNK
        pltpu.sync_copy(pos_hbm.at[pl.ds(base, CHUNK)], pos_v)
        for k in range(CHUNK // L):
            p = pos_v[pl.ds(k * L, L)]
            ix = jnp.minimum(jnp.maximum(p * float(G - 1), 0.0),
                             float(G - 1))
            i0 = ix.astype(jnp.int32)
            w = ix - i0.astype(jnp.float32)
            i1 = jnp.minimum(i0 + 1, G - 1)
            idx0_v[pl.ds(k * L, L)] = i0 + b_off
            idx1_v[pl.ds(k * L, L)] = i1 + b_off
            w_v[pl.ds(k * L, L)] = w
        cp0 = pltpu.async_copy(field_hbm.at[idx0_v], f0_v, sem0)
        cp1 = pltpu.async_copy(field_hbm.at[idx1_v], f1_v, sem1)
        cp0.wait()
        cp1.wait()

        def group_body(g, carry):
            wvec = w_v[pl.ds(g * L, L)]
            for s in range(L):
                wb = jnp.broadcast_to(wvec[s], (L,))
                r = g * L + s
                for j in range(D // L):
                    a = f0_v[r, pl.ds(j * L, L)]
                    b = f1_v[r, pl.ds(j * L, L)]
                    o_v[r, pl.ds(j * L, L)] = a + wb * (b - a)
            return carry

        lax.fori_loop(0, CHUNK // L, group_body, 0)
        pltpu.sync_copy(o_v, out_hbm.at[pl.ds(base, CHUNK)])
        return _

    lax.fori_loop(0, NCHUNK, chunk_body, 0)


def kernel(field, grid_points, sample_positions):
    del grid_points
    field2d = field.reshape(B * G, D)
    pos_flat = sample_positions.reshape(TOTAL)
    mesh = plsc.VectorSubcoreMesh(core_axis_name="c", subcore_axis_name="s",
                                  num_cores=NC, num_subcores=NS)
    out2d = pl.kernel(
        _sc_body,
        out_type=jax.ShapeDtypeStruct((TOTAL, D), jnp.float32),
        mesh=mesh,
        scratch_types=[
            pltpu.VMEM((CHUNK,), jnp.float32),
            pltpu.VMEM((CHUNK,), jnp.float32),
            pltpu.VMEM((CHUNK,), jnp.int32),
            pltpu.VMEM((CHUNK,), jnp.int32),
            pltpu.VMEM((CHUNK, D), jnp.float32),
            pltpu.VMEM((CHUNK, D), jnp.float32),
            pltpu.VMEM((CHUNK, D), jnp.float32),
            pltpu.SemaphoreType.DMA,
            pltpu.SemaphoreType.DMA,
        ],
    )(field2d, pos_flat)
    return out2d.reshape(B, N, D)


# speedup vs baseline: 7181.8673x; 1.0259x over previous
import jax
import jax.numpy as jnp
from jax import lax
from jax.experimental import pallas as pl
from jax.experimental.pallas import tpu as pltpu
from jax.experimental.pallas import tpu_sc as plsc

B, G, D, N = 16, 4096, 128, 16384
NC, NS, L = 2, 16, 16
NW = NC * NS
TOTAL = B * N
PER_W = TOTAL // NW
CHUNK = 128
NCHUNK = PER_W // CHUNK
NBUF = 2


def _sc_body(field_hbm, pos_hbm, out_hbm,
             pos_all, w0, w1, i0a, i0b, i1a, i1b,
             f0a, f0b, f1a, f1b, oa, ob,
             gs0, gs1, ss0, ss1):
    w_v = (w0, w1)
    idx0_v = (i0a, i0b)
    idx1_v = (i1a, i1b)
    f0_v = (f0a, f0b)
    f1_v = (f1a, f1b)
    o_v = (oa, ob)
    gsem = (gs0, gs1)
    ssem = (ss0, ss1)

    wid = lax.axis_index("s") * NC + lax.axis_index("c")
    wbase = wid * PER_W
    b_off = (wbase // N) * G

    pltpu.sync_copy(pos_hbm.at[pl.ds(wbase, PER_W)], pos_all)

    def fire_gather(c, par):
        for k in range(CHUNK // L):
            p = pos_all[pl.ds(c * CHUNK + k * L, L)]
            ix = jnp.minimum(jnp.maximum(p * float(G - 1), 0.0),
                             float(G - 1))
            i0 = ix.astype(jnp.int32)
            w = ix - i0.astype(jnp.float32)
            i1 = jnp.minimum(i0 + 1, G - 1)
            idx0_v[par][pl.ds(k * L, L)] = i0 + b_off
            idx1_v[par][pl.ds(k * L, L)] = i1 + b_off
            w_v[par][pl.ds(k * L, L)] = w
        pltpu.async_copy(field_hbm.at[idx0_v[par]], f0_v[par], gsem[par])
        pltpu.async_copy(field_hbm.at[idx1_v[par]], f1_v[par], gsem[par])

    def wait_gather(par):
        pltpu.make_async_copy(field_hbm.at[idx0_v[par]], f0_v[par],
                              gsem[par]).wait()
        pltpu.make_async_copy(field_hbm.at[idx1_v[par]], f1_v[par],
                              gsem[par]).wait()

    def lerp(par):
        def group_body(g, carry):
            wvec = w_v[par][pl.ds(g * L, L)]
            for s in range(L):
                wb = jnp.broadcast_to(wvec[s], (L,))
                r = g * L + s
                for j in range(D // L):
                    a = f0_v[par][r, pl.ds(j * L, L)]
                    b = f1_v[par][r, pl.ds(j * L, L)]
                    o_v[par][r, pl.ds(j * L, L)] = a + wb * (b - a)
            return carry

        lax.fori_loop(0, CHUNK // L, group_body, 0)

    def fire_scatter(c, par):
        pltpu.async_copy(o_v[par], out_hbm.at[pl.ds(wbase + c * CHUNK, CHUNK)],
                         ssem[par])

    def wait_scatter(par):
        pltpu.make_async_copy(o_v[par], out_hbm.at[pl.ds(0, CHUNK)],
                              ssem[par]).wait()

    fire_gather(0, 0)
    fire_gather(1, 1)

    def pair_body(cc, carry):
        for par in range(NBUF):
            c = NBUF * cc + par
            wait_gather(par)
            pl.when(cc > 0)(lambda: wait_scatter(par))
            lerp(par)
            fire_scatter(c, par)
            fire_gather(jnp.minimum(c + NBUF, NCHUNK - 1), par)
        return carry

    lax.fori_loop(0, NCHUNK // NBUF, pair_body, 0)

    for par in range(NBUF):
        wait_gather(par)
        wait_scatter(par)


def kernel(field, grid_points, sample_positions):
    del grid_points
    field2d = field.reshape(B * G, D)
    pos_flat = sample_positions.reshape(TOTAL)
    mesh = plsc.VectorSubcoreMesh(core_axis_name="c", subcore_axis_name="s",
                                  num_cores=NC, num_subcores=NS)
    out2d = pl.kernel(
        _sc_body,
        out_type=jax.ShapeDtypeStruct((TOTAL, D), jnp.float32),
        mesh=mesh,
        scratch_types=(
            [pltpu.VMEM((PER_W,), jnp.float32)]
            + [pltpu.VMEM((CHUNK,), jnp.float32)] * 2
            + [pltpu.VMEM((CHUNK,), jnp.int32)] * 4
            + [pltpu.VMEM((CHUNK, D), jnp.float32)] * 6
            + [pltpu.SemaphoreType.DMA] * 4
        ),
    )(field2d, pos_flat)
    return out2d.reshape(B, N, D)


# speedup vs baseline: 16615.2700x; 2.3135x over previous
import jax
import jax.numpy as jnp
from jax import lax
from jax.experimental import pallas as pl
from jax.experimental.pallas import tpu as pltpu
from jax.experimental.pallas import tpu_sc as plsc

B, G, D, N = 16, 4096, 128, 16384
NC, NS, L = 2, 16, 16
NW = NC * NS
TOTAL = B * N
PER_W = TOTAL // NW
CHUNK = 64
NCHUNK = PER_W // CHUNK
NBUF = 2


def _sc_body(field_hbm, pos_hbm, out_hbm,
             pa, pb, w0, w1, i0a, i0b, i1a, i1b,
             f0a, f0b, f1a, f1b, oa, ob,
             gs0, gs1, ss0, ss1):
    pos_v = (pa, pb)
    w_v = (w0, w1)
    idx0_v = (i0a, i0b)
    idx1_v = (i1a, i1b)
    f0_v = (f0a, f0b)
    f1_v = (f1a, f1b)
    o_v = (oa, ob)
    gsem = (gs0, gs1)
    ssem = (ss0, ss1)

    wid = lax.axis_index("s") * NC + lax.axis_index("c")
    wbase = wid * PER_W
    b_off = (wbase // N) * G

    def fire_gather(c, par):
        pltpu.sync_copy(pos_hbm.at[pl.ds(wbase + c * CHUNK, CHUNK)],
                        pos_v[par])
        for k in range(CHUNK // L):
            p = pos_v[par][pl.ds(k * L, L)]
            ix = jnp.minimum(jnp.maximum(p * float(G - 1), 0.0),
                             float(G - 1))
            i0 = ix.astype(jnp.int32)
            w = ix - i0.astype(jnp.float32)
            i1 = jnp.minimum(i0 + 1, G - 1)
            idx0_v[par][pl.ds(k * L, L)] = i0 + b_off
            idx1_v[par][pl.ds(k * L, L)] = i1 + b_off
            for s in range(L):
                w_v[par][k * L + s, :] = jnp.broadcast_to(w[s], (L,))
        pltpu.async_copy(field_hbm.at[idx0_v[par]], f0_v[par], gsem[par])
        pltpu.async_copy(field_hbm.at[idx1_v[par]], f1_v[par], gsem[par])

    def wait_gather(par):
        pltpu.make_async_copy(field_hbm.at[idx0_v[par]], f0_v[par],
                              gsem[par]).wait()
        pltpu.make_async_copy(field_hbm.at[idx1_v[par]], f1_v[par],
                              gsem[par]).wait()

    def lerp(par):
        def sample_body(s, carry):
            wb = w_v[par][s, :]
            for j in range(D // L):
                a = f0_v[par][s, pl.ds(j * L, L)]
                b = f1_v[par][s, pl.ds(j * L, L)]
                o_v[par][s, pl.ds(j * L, L)] = a + wb * (b - a)
            return carry

        lax.fori_loop(0, CHUNK, sample_body, 0)

    def fire_scatter(c, par):
        pltpu.async_copy(o_v[par], out_hbm.at[pl.ds(wbase + c * CHUNK, CHUNK)],
                         ssem[par])

    def wait_scatter(par):
        pltpu.make_async_copy(o_v[par], out_hbm.at[pl.ds(0, CHUNK)],
                              ssem[par]).wait()

    fire_gather(0, 0)
    fire_gather(1, 1)

    def pair_body(cc, carry):
        for par in range(NBUF):
            c = NBUF * cc + par
            wait_gather(par)
            pl.when(cc > 0)(lambda: wait_scatter(par))
            lerp(par)
            fire_scatter(c, par)
            fire_gather(jnp.minimum(c + NBUF, NCHUNK - 1), par)
        return carry

    lax.fori_loop(0, NCHUNK // NBUF, pair_body, 0)

    for par in range(NBUF):
        wait_gather(par)
        wait_scatter(par)


def kernel(field, grid_points, sample_positions):
    del grid_points
    field2d = field.reshape(B * G, D)
    pos_flat = sample_positions.reshape(TOTAL)
    mesh = plsc.VectorSubcoreMesh(core_axis_name="c", subcore_axis_name="s",
                                  num_cores=NC, num_subcores=NS)
    out2d = pl.kernel(
        _sc_body,
        out_type=jax.ShapeDtypeStruct((TOTAL, D), jnp.float32),
        mesh=mesh,
        scratch_types=(
            [pltpu.VMEM((CHUNK,), jnp.float32)] * 2
            + [pltpu.VMEM((CHUNK, L), jnp.float32)] * 2
            + [pltpu.VMEM((CHUNK,), jnp.int32)] * 4
            + [pltpu.VMEM((CHUNK, D), jnp.float32)] * 6
            + [pltpu.SemaphoreType.DMA] * 4
        ),
    )(field2d, pos_flat)
    return out2d.reshape(B, N, D)


# speedup vs baseline: 18298.3965x; 1.1013x over previous
import jax
import jax.numpy as jnp
from jax import lax
from jax.experimental import pallas as pl
from jax.experimental.pallas import tpu as pltpu
from jax.experimental.pallas import tpu_sc as plsc

B, G, D, N = 16, 4096, 128, 16384
NC, NS, L = 2, 16, 16
NW = NC * NS
TOTAL = B * N
PER_W = TOTAL // NW
CHUNK = 64
NCHUNK = PER_W // CHUNK
NBUF = 3
NFULL = NCHUNK // NBUF
TAIL = NCHUNK - NFULL * NBUF


def _sc_body(field_hbm, pos_hbm, out_hbm, *scr):
    pos_v = scr[0:NBUF]
    w_v = scr[NBUF:2 * NBUF]
    idx0_v = scr[2 * NBUF:3 * NBUF]
    idx1_v = scr[3 * NBUF:4 * NBUF]
    f0_v = scr[4 * NBUF:5 * NBUF]
    f1_v = scr[5 * NBUF:6 * NBUF]
    o_v = scr[6 * NBUF:7 * NBUF]
    gsem = scr[7 * NBUF:8 * NBUF]
    ssem = scr[8 * NBUF:9 * NBUF]

    wid = lax.axis_index("s") * NC + lax.axis_index("c")
    wbase = wid * PER_W
    b_off = (wbase // N) * G

    def fire_gather(c, par):
        pltpu.sync_copy(pos_hbm.at[pl.ds(wbase + c * CHUNK, CHUNK)],
                        pos_v[par])
        for k in range(CHUNK // L):
            p = pos_v[par][pl.ds(k * L, L)]
            ix = jnp.minimum(jnp.maximum(p * float(G - 1), 0.0),
                             float(G - 1))
            i0 = ix.astype(jnp.int32)
            w = ix - i0.astype(jnp.float32)
            i1 = jnp.minimum(i0 + 1, G - 1)
            idx0_v[par][pl.ds(k * L, L)] = i0 + b_off
            idx1_v[par][pl.ds(k * L, L)] = i1 + b_off
            for s in range(L):
                w_v[par][k * L + s, :] = jnp.broadcast_to(w[s], (L,))
        pltpu.async_copy(field_hbm.at[idx0_v[par]], f0_v[par], gsem[par])
        pltpu.async_copy(field_hbm.at[idx1_v[par]], f1_v[par], gsem[par])

    def wait_gather(par):
        pltpu.make_async_copy(field_hbm.at[idx0_v[par]], f0_v[par],
                              gsem[par]).wait()
        pltpu.make_async_copy(field_hbm.at[idx1_v[par]], f1_v[par],
                              gsem[par]).wait()

    def lerp(par):
        def sample_body(s, carry):
            wb = w_v[par][s, :]
            for j in range(D // L):
                a = f0_v[par][s, pl.ds(j * L, L)]
                b = f1_v[par][s, pl.ds(j * L, L)]
                o_v[par][s, pl.ds(j * L, L)] = a + wb * (b - a)
            return carry

        lax.fori_loop(0, CHUNK, sample_body, 0)

    def fire_scatter(c, par):
        pltpu.async_copy(o_v[par], out_hbm.at[pl.ds(wbase + c * CHUNK, CHUNK)],
                         ssem[par])

    def wait_scatter(par):
        pltpu.make_async_copy(o_v[par], out_hbm.at[pl.ds(0, CHUNK)],
                              ssem[par]).wait()

    for par in range(NBUF):
        fire_gather(par, par)

    def rot_body(cc, carry):
        for par in range(NBUF):
            c = NBUF * cc + par
            wait_gather(par)
            pl.when(cc > 0)(lambda par=par: wait_scatter(par))
            lerp(par)
            fire_scatter(c, par)
            pl.when(c + NBUF < NCHUNK)(
                lambda c=c, par=par: fire_gather(c + NBUF, par))
        return carry

    lax.fori_loop(0, NFULL, rot_body, 0)

    for par in range(TAIL):
        c = NFULL * NBUF + par
        wait_gather(par)
        wait_scatter(par)
        lerp(par)
        fire_scatter(c, par)

    for par in range(NBUF):
        wait_scatter(par)


def kernel(field, grid_points, sample_positions):
    del grid_points
    field2d = field.reshape(B * G, D)
    pos_flat = sample_positions.reshape(TOTAL)
    mesh = plsc.VectorSubcoreMesh(core_axis_name="c", subcore_axis_name="s",
                                  num_cores=NC, num_subcores=NS)
    out2d = pl.kernel(
        _sc_body,
        out_type=jax.ShapeDtypeStruct((TOTAL, D), jnp.float32),
        mesh=mesh,
        scratch_types=(
            [pltpu.VMEM((CHUNK,), jnp.float32)] * NBUF
            + [pltpu.VMEM((CHUNK, L), jnp.float32)] * NBUF
            + [pltpu.VMEM((CHUNK,), jnp.int32)] * NBUF
            + [pltpu.VMEM((CHUNK,), jnp.int32)] * NBUF
            + [pltpu.VMEM((CHUNK, D), jnp.float32)] * NBUF
            + [pltpu.VMEM((CHUNK, D), jnp.float32)] * NBUF
            + [pltpu.VMEM((CHUNK, D), jnp.float32)] * NBUF
            + [pltpu.SemaphoreType.DMA] * NBUF
            + [pltpu.SemaphoreType.DMA] * NBUF
        ),
    )(field2d, pos_flat)
    return out2d.reshape(B, N, D)
